# Initial kernel scaffold; baseline (speedup 1.0000x reference)
import jax
import jax.numpy as jnp
from jax.experimental import pallas as pl


def kernel(x, edge_index, W1, b1, W2, b2, W3, b3, Wl, bl):
    raise NotImplementedError("write your pallas kernel here")


# speedup vs baseline: 5.3752x; 5.3752x over previous
import jax
import jax.numpy as jnp
from jax import lax
from jax.experimental import pallas as pl
from jax.experimental.pallas import tpu as pltpu
from jax.experimental.pallas import tpu_sc as plsc

N = 10000
E = 320000
D = 128
NP = 10240
NC = 2
NS = 16
NW = NC * NS
EPW = E // NW
CS = 128
CH = NP // CS
RPT = NP // NS
KO = RPT // CS
PAD_SRC = N
PAD_DST = NP - 1

_mesh = plsc.VectorSubcoreMesh(core_axis_name="c", subcore_axis_name="s",
                               num_cores=NC, num_subcores=NS)


def _row_fill(ref, n, valfn):
    def f(i, _):
        for j16 in range(D // 16):
            ref[i, pl.ds(j16 * 16, 16)] = valfn(i, j16)
        return 0
    lax.fori_loop(0, n, f, 0)


def _fill_own_idx(idx_ref, s, k):
    base = s * RPT + k * CS
    for j16 in range(CS // 16):
        idx_ref[pl.ds(j16 * 16, 16)] = (base + j16 * 16
                                        + lax.iota(jnp.int32, 16))


def _edge_body(hs_hbm, src_hbm, dst_hbm, agg_out, rows, zbuf,
               sidx, didx, acc, gsem, ssem):
    c = lax.axis_index("c")
    s = lax.axis_index("s")
    w = c * NS + s
    _row_fill(zbuf, CS, lambda i, j16: jnp.zeros((16,), jnp.float32))
    for k in range(KO):
        _fill_own_idx(didx, s, k)
        pltpu.async_copy(zbuf, acc.at[didx], ssem).wait()
    plsc.subcore_barrier()

    def body(j, _):
        pltpu.sync_copy(src_hbm.at[w, j], sidx)
        pltpu.sync_copy(dst_hbm.at[w, j], didx)
        pltpu.async_copy(hs_hbm.at[sidx], rows, gsem).wait()
        pltpu.async_copy(rows, acc.at[didx], ssem, add=True).wait()
        return 0
    lax.fori_loop(0, CH, body, 0)

    plsc.subcore_barrier()
    for k in range(KO):
        _fill_own_idx(didx, s, k)
        pltpu.async_copy(acc.at[didx], zbuf, gsem).wait()
        pltpu.sync_copy(zbuf, agg_out.at[c, pl.ds(s * RPT + k * CS, CS)])


_edge_kernel = pl.kernel(
    _edge_body,
    out_type=jax.ShapeDtypeStruct((NC, NP, D), jnp.float32),
    mesh=_mesh,
    scratch_types=[
        pltpu.VMEM((CS, D), jnp.float32),
        pltpu.VMEM((CS, D), jnp.float32),
        pltpu.VMEM((CS,), jnp.int32),
        pltpu.VMEM((CS,), jnp.int32),
        pltpu.VMEM_SHARED((NP, D), jnp.float32),
        pltpu.SemaphoreType.DMA,
        pltpu.SemaphoreType.DMA,
    ],
)


def _dis_from_deg(deg_blk):
    counts = deg_blk[0, :, 0] + deg_blk[1, :, 0] + 1.0
    return lax.rsqrt(counts)


def _b1_body(x_ref, w_ref, deg_ref, out_ref):
    dis = _dis_from_deg(deg_ref[...])
    h = jnp.dot(x_ref[...], w_ref[...], preferred_element_type=jnp.float32)
    out_ref[...] = h * dis[:, None]


def _bn_body(agg_ref, hs_ref, deg_ref, b_ref, w_ref, out_ref):
    dis = _dis_from_deg(deg_ref[...])
    pre = dis[:, None] * (agg_ref[0] + agg_ref[1] + hs_ref[...]) + b_ref[...]
    h = jnp.maximum(pre, 0.0)
    h = jnp.dot(h, w_ref[...], preferred_element_type=jnp.float32)
    out_ref[...] = h * dis[:, None]


def _fin_body(agg_ref, hs_ref, deg_ref, b_ref, wl_ref, bl_ref, out_ref):
    dis = _dis_from_deg(deg_ref[...])
    pre = dis[:, None] * (agg_ref[0] + agg_ref[1] + hs_ref[...]) + b_ref[...]
    h = jnp.maximum(pre, 0.0)
    out_ref[...] = (
        jnp.dot(h, wl_ref[...], preferred_element_type=jnp.float32)
        + bl_ref[...]
    )


_BLK = 128
_GRID = NP // _BLK

_spec_rows = pl.BlockSpec((_BLK, D), lambda i: (i, 0))
_spec_w = pl.BlockSpec((D, D), lambda i: (0, 0))
_spec_agg = pl.BlockSpec((NC, _BLK, D), lambda i: (0, i, 0))
_spec_b = pl.BlockSpec((1, D), lambda i: (0, 0))

_b1_call = pl.pallas_call(
    _b1_body,
    grid=(_GRID,),
    in_specs=[_spec_rows, _spec_w, _spec_agg],
    out_specs=_spec_rows,
    out_shape=jax.ShapeDtypeStruct((NP, D), jnp.float32),
)

_bn_call = pl.pallas_call(
    _bn_body,
    grid=(_GRID,),
    in_specs=[_spec_agg, _spec_rows, _spec_agg, _spec_b, _spec_w],
    out_specs=_spec_rows,
    out_shape=jax.ShapeDtypeStruct((NP, D), jnp.float32),
)

_fin_call = pl.pallas_call(
    _fin_body,
    grid=(1,),
    in_specs=[
        pl.BlockSpec((NC, 8, D), lambda i: (0, 0, 0)),
        pl.BlockSpec((8, D), lambda i: (0, 0)),
        pl.BlockSpec((NC, 8, D), lambda i: (0, 0, 0)),
        _spec_b,
        _spec_w,
        _spec_b,
    ],
    out_specs=pl.BlockSpec((8, D), lambda i: (0, 0)),
    out_shape=jax.ShapeDtypeStruct((8, D), jnp.float32),
)


def kernel(x, edge_index, W1, b1, W2, b2, W3, b3, Wl, bl):
    x_pad = jnp.pad(x, ((0, NP - N), (0, 0)))
    src = edge_index[0].reshape(NW, EPW)
    dst = edge_index[1].reshape(NW, EPW)
    pad = NP - EPW
    src_pad = jnp.pad(src, ((0, 0), (0, pad)),
                      constant_values=PAD_SRC).reshape(NW, CH, CS)
    dst_pad = jnp.pad(dst, ((0, 0), (0, pad)),
                      constant_values=PAD_DST).reshape(NW, CH, CS)
    b1r = b1.reshape(1, D)
    b2r = b2.reshape(1, D)
    b3r = b3.reshape(1, D)
    wl_pad = jnp.pad(Wl, ((0, 0), (0, D - Wl.shape[1])))
    bl_pad = jnp.pad(bl, ((0, D - bl.shape[0]),)).reshape(1, D)

    ones_table = jnp.ones((NP, D), jnp.float32)
    deg = _edge_kernel(ones_table, src_pad, dst_pad)

    hs1 = _b1_call(x_pad, W1, deg)
    agg1 = _edge_kernel(hs1, src_pad, dst_pad)
    hs2 = _bn_call(agg1, hs1, deg, b1r, W2)
    agg2 = _edge_kernel(hs2, src_pad, dst_pad)
    hs3 = _bn_call(agg2, hs2, deg, b2r, W3)
    agg3 = _edge_kernel(hs3, src_pad, dst_pad)
    out = _fin_call(agg3, hs3, deg, b3r, wl_pad, bl_pad)
    return out[:5, :3]
